# Initial kernel scaffold; baseline (speedup 1.0000x reference)
import jax
import jax.numpy as jnp
from jax.experimental import pallas as pl


def kernel(x, edge_index, W1, b1, Wg, bg, W2, b2, Wg2, bg2, W3, b3):
    raise NotImplementedError("write your pallas kernel here")


# speedup vs baseline: 7.5550x; 7.5550x over previous
import functools

import jax
import jax.numpy as jnp
from jax import lax
from jax.experimental import pallas as pl
from jax.experimental.pallas import tpu as pltpu
from jax.experimental.pallas import tpu_sc as plsc

N = 10000
E = 320000
NC = 2
NS = 16
NW = NC * NS
EPW = E // NW
CHUNK = 80
NCHUNK = EPW // CHUNK
RPT = N // NS

_F32 = jnp.float32


def _fill2d(ref, rows, cols, value):
    v = jnp.full((16,), value, _F32)
    nb = cols // 16

    def body(r, _):
        for k in range(nb):
            ref[r, pl.ds(16 * k, 16)] = v
        return 0

    lax.fori_loop(0, rows, body, 0)


def _make_spmm(passes):
    mesh = plsc.VectorSubcoreMesh(core_axis_name="c", subcore_axis_name="s")
    out_type = [jax.ShapeDtypeStruct((NC, N, dw), _F32)
                for (dw, _, _) in passes]
    scratch = [
        pltpu.VMEM((NCHUNK, CHUNK), jnp.int32),
        pltpu.VMEM((NCHUNK, CHUNK), jnp.int32),
        pltpu.SemaphoreType.DMA,
    ]
    need_ones = any(not g for (_, g, _) in passes)
    if need_ones:
        scratch.append(pltpu.VMEM((CHUNK, 16), _F32))
    for dw, gather, _ in passes:
        if gather:
            scratch.append(pltpu.VMEM((CHUNK, dw), _F32))
        scratch.append(pltpu.VMEM((NCHUNK, dw), _F32))
        scratch.append(pltpu.VMEM_SHARED((N, dw), _F32))

    n_tables = sum(1 for (_, g, _) in passes if g)

    @functools.partial(
        pl.kernel, out_type=tuple(out_type), mesh=mesh,
        scratch_types=tuple(scratch),
        compiler_params=pltpu.CompilerParams(use_tc_tiling_on_sc=False))
    def body(*refs):
        tables = refs[:n_tables]
        srcr_hbm, dstr_hbm = refs[n_tables:n_tables + 2]
        outs = refs[n_tables + 2:n_tables + 2 + len(passes)]
        rest = list(refs[n_tables + 2 + len(passes):])
        src_v = rest.pop(0)
        dst_v = rest.pop(0)
        sem = rest.pop(0)
        ones_v = rest.pop(0) if need_ones else None
        rows_vs, zbs, accs = [], [], []
        for dw, gather, _ in passes:
            rows_vs.append(rest.pop(0) if gather else ones_v)
            zbs.append(rest.pop(0))
            accs.append(rest.pop(0))

        c = lax.axis_index("c")
        s = lax.axis_index("s")
        w = s * NC + c

        pltpu.sync_copy(srcr_hbm.at[w], src_v)
        pltpu.sync_copy(dstr_hbm.at[w], dst_v)
        if need_ones:
            _fill2d(ones_v, CHUNK, 16, 1.0)
        for (dw, _, _), zb, acc in zip(passes, zbs, accs):
            _fill2d(zb, NCHUNK, dw, 0.0)
            for k in range(RPT // NCHUNK):
                pltpu.sync_copy(zb, acc.at[pl.ds(s * RPT + k * NCHUNK,
                                                 NCHUNK)])
        plsc.subcore_barrier()

        def chunk(j, _):
            ti = 0
            for (dw, gather, by_src), rows_v, acc in zip(
                    passes, rows_vs, accs):
                if gather:
                    pltpu.async_copy(
                        tables[ti].at[src_v.at[j]], rows_v, sem).wait()
                    ti += 1
                key = src_v.at[j] if by_src else dst_v.at[j]
                pltpu.sync_copy(rows_v, acc.at[key], add=True)
            return 0

        lax.fori_loop(0, NCHUNK, chunk, 0)
        plsc.subcore_barrier()

        for (dw, _, _), acc, out in zip(passes, accs, outs):
            pltpu.sync_copy(acc.at[pl.ds(s * RPT, RPT)],
                            out.at[c].at[pl.ds(s * RPT, RPT)])

    return body


_deg_kernel = _make_spmm(((16, False, False),
                          (16, False, True)))
_spmm128 = _make_spmm(((128, True, False),))
_spmm64_c4 = _make_spmm(((64, True, False),
                         (16, True, False)))
_spmm64 = _make_spmm(((64, True, False),))


def _leaky(v):
    return jnp.where(v >= 0, v, 0.01 * v)


def _t1_body(ya, yb, dia, dib, doa, dob, W1, b1, Wg, bg,
             feat_o, rsqout16_o, invdeg_o, rsqin_o):
    deg_in = jnp.maximum(dia[:, 0:1] + dib[:, 0:1], 1.0)
    deg_out = jnp.maximum(doa[:, 0:1] + dob[:, 0:1], 1.0)
    invdeg = 1.0 / deg_in
    rsq_in = jnp.sqrt(invdeg)
    rsq_out = lax.rsqrt(deg_out)
    m1 = (ya[...] + yb[...]) * invdeg
    h1 = _leaky(jnp.dot(m1, W1[...],
                        preferred_element_type=_F32) + b1[...])
    feat_o[...] = jnp.dot(h1, Wg[...],
                          preferred_element_type=_F32) * rsq_out
    rsqout16_o[...] = jnp.broadcast_to(rsq_out, (N, 16))
    invdeg_o[...] = invdeg
    rsqin_o[...] = rsq_in


_t1 = pl.pallas_call(
    _t1_body,
    out_shape=(jax.ShapeDtypeStruct((N, 128), _F32),
               jax.ShapeDtypeStruct((N, 16), _F32),
               jax.ShapeDtypeStruct((N, 1), _F32),
               jax.ShapeDtypeStruct((N, 1), _F32)),
)


def _t2_body(ya, yb, rsqin, bg, W2, g3_o):
    agg2 = (ya[...] + yb[...]) * rsqin[...] + bg[...]
    h2 = _leaky(agg2)
    g3_o[...] = jnp.dot(h2, W2[...], preferred_element_type=_F32)


_t2 = pl.pallas_call(
    _t2_body,
    out_shape=jax.ShapeDtypeStruct((N, 64), _F32),
)


def _t3_body(ya, yb, c4a, c4b, invdeg, rsqin, b2, Wg2, bg2, W3, g5_o):
    h3 = (ya[...] + yb[...]) * invdeg[...] + b2[...]
    pooled = jnp.mean(h3, axis=0, keepdims=True)
    q = jnp.dot(pooled, Wg2[...], preferred_element_type=_F32)
    alpha = (c4a[:, 0:1] + c4b[:, 0:1]) * rsqin[...]
    h4 = _leaky(alpha * q + bg2[...])
    g5_o[...] = jnp.dot(h4, W3[...], preferred_element_type=_F32)


_t3 = pl.pallas_call(
    _t3_body,
    out_shape=jax.ShapeDtypeStruct((N, 64), _F32),
)


def _t4_body(ya, yb, invdeg, b3, out_o):
    out_o[...] = (ya[...] + yb[...]) * invdeg[...] + b3[...]


_t4 = pl.pallas_call(
    _t4_body,
    out_shape=jax.ShapeDtypeStruct((N, 64), _F32),
)


def kernel(x, edge_index, W1, b1, Wg, bg, W2, b2, Wg2, bg2, W3, b3):
    src_r = edge_index[0].reshape(NW, NCHUNK, CHUNK)
    dst_r = edge_index[1].reshape(NW, NCHUNK, CHUNK)

    din, dout = _deg_kernel(src_r, dst_r)
    y1 = _spmm128(x, src_r, dst_r)[0]
    feat2, rsqout16, invdeg, rsqin = _t1(
        y1[0], y1[1], din[0], din[1], dout[0], dout[1], W1, b1, Wg, bg)
    y2 = _spmm128(feat2, src_r, dst_r)[0]
    g3 = _t2(y2[0], y2[1], rsqin, bg, W2)
    y3, c4 = _spmm64_c4(g3, rsqout16, src_r, dst_r)
    g5 = _t3(y3[0], y3[1], c4[0], c4[1], invdeg, rsqin, b2, Wg2, bg2, W3)
    y5 = _spmm64(g5, src_r, dst_r)[0]
    out = _t4(y5[0], y5[1], invdeg, b3)
    return out


# speedup vs baseline: 14.7121x; 1.9473x over previous
import functools

import jax
import jax.numpy as jnp
from jax import lax
from jax.experimental import pallas as pl
from jax.experimental.pallas import tpu as pltpu
from jax.experimental.pallas import tpu_sc as plsc

N = 10000
E = 320000
NC = 2
NS = 16
NW = NC * NS
EPW = E // NW
CHUNK = 40
NCHUNK = EPW // CHUNK
RPT = N // NS
NBUF = 5
ZBR = 25

_F32 = jnp.float32


def _fill2d(ref, rows, cols, value):
    v = jnp.full((16,), value, _F32)
    nb = cols // 16

    def body(r, _):
        for k in range(nb):
            ref[r, pl.ds(16 * k, 16)] = v
        return 0

    lax.fori_loop(0, rows, body, 0)


def _zero_acc(zb, acc, s):
    _fill2d(zb, ZBR, zb.shape[1], 0.0)

    def body(k, _):
        pltpu.sync_copy(zb, acc.at[pl.ds(s * RPT + k * ZBR, ZBR)])
        return 0

    lax.fori_loop(0, RPT // ZBR, body, 0)


def _copy_out(acc, out, c, s):
    pltpu.sync_copy(acc.at[pl.ds(s * RPT, RPT)],
                    out.at[c].at[pl.ds(s * RPT, RPT)])


def _make_gspmm(dw):
    mesh = plsc.VectorSubcoreMesh(core_axis_name="c", subcore_axis_name="s")
    scratch = [
        pltpu.VMEM((NCHUNK, CHUNK), jnp.int32),
        pltpu.VMEM((NCHUNK, CHUNK), jnp.int32),
        pltpu.VMEM((ZBR, dw), _F32),
        pltpu.VMEM_SHARED((N, dw), _F32),
    ]
    scratch += [pltpu.VMEM((CHUNK, dw), _F32) for _ in range(NBUF)]
    scratch += [pltpu.SemaphoreType.DMA for _ in range(NBUF)]

    @functools.partial(
        pl.kernel, out_type=jax.ShapeDtypeStruct((NC, N, dw), _F32),
        mesh=mesh, scratch_types=tuple(scratch),
        compiler_params=pltpu.CompilerParams(use_tc_tiling_on_sc=False))
    def body(g_hbm, srcr_hbm, dstr_hbm, y_out, src_v, dst_v, zb, acc,
             *bufsem):
        bufs = bufsem[:NBUF]
        sems = bufsem[NBUF:]
        c = lax.axis_index("c")
        s = lax.axis_index("s")
        w = s * NC + c

        pltpu.sync_copy(srcr_hbm.at[w], src_v)
        pltpu.sync_copy(dstr_hbm.at[w], dst_v)
        _zero_acc(zb, acc, s)
        plsc.subcore_barrier()

        def fire(j, b):
            pltpu.async_copy(g_hbm.at[src_v.at[j]], bufs[b], sems[b])

        def drain_and_scatter(j, b):
            pltpu.make_async_copy(
                g_hbm.at[src_v.at[j]], bufs[b], sems[b]).wait()
            pltpu.sync_copy(bufs[b], acc.at[dst_v.at[j]], add=True)

        for b in range(NBUF):
            fire(b, b)

        def group(gi, _):
            j0 = gi * NBUF
            for b in range(NBUF):
                drain_and_scatter(j0 + b, b)
                fire(j0 + b + NBUF, b)
            return 0

        lax.fori_loop(0, NCHUNK // NBUF - 1, group, 0)
        for b in range(NBUF):
            drain_and_scatter(NCHUNK - NBUF + b, b)

        plsc.subcore_barrier()
        _copy_out(acc, y_out, c, s)

    return body


def _make_deg():
    mesh = plsc.VectorSubcoreMesh(core_axis_name="c", subcore_axis_name="s")
    scratch = [
        pltpu.VMEM((NCHUNK, CHUNK), jnp.int32),
        pltpu.VMEM((NCHUNK, CHUNK), jnp.int32),
        pltpu.VMEM((CHUNK, 16), _F32),
        pltpu.VMEM((ZBR, 16), _F32),
        pltpu.VMEM_SHARED((N, 16), _F32),
        pltpu.VMEM_SHARED((N, 16), _F32),
        pltpu.SemaphoreType.DMA,
        pltpu.SemaphoreType.DMA,
    ]

    @functools.partial(
        pl.kernel,
        out_type=(jax.ShapeDtypeStruct((NC, N, 16), _F32),
                  jax.ShapeDtypeStruct((NC, N, 16), _F32)),
        mesh=mesh, scratch_types=tuple(scratch),
        compiler_params=pltpu.CompilerParams(use_tc_tiling_on_sc=False))
    def body(srcr_hbm, dstr_hbm, din_out, dout_out, src_v, dst_v,
             ones_v, zb, din, dout, sem1, sem2):
        c = lax.axis_index("c")
        s = lax.axis_index("s")
        w = s * NC + c

        pltpu.sync_copy(srcr_hbm.at[w], src_v)
        pltpu.sync_copy(dstr_hbm.at[w], dst_v)
        _fill2d(ones_v, CHUNK, 16, 1.0)
        _zero_acc(zb, din, s)
        _zero_acc(zb, dout, s)
        plsc.subcore_barrier()

        def chunk(j, _):
            d1 = pltpu.async_copy(ones_v, din.at[dst_v.at[j]], sem1,
                                  add=True)
            d2 = pltpu.async_copy(ones_v, dout.at[src_v.at[j]], sem2,
                                  add=True)
            d1.wait()
            d2.wait()
            return 0

        lax.fori_loop(0, NCHUNK, chunk, 0)
        plsc.subcore_barrier()
        _copy_out(din, din_out, c, s)
        _copy_out(dout, dout_out, c, s)

    return body


_deg_kernel = _make_deg()
_spmm128 = _make_gspmm(128)
_spmm80 = _make_gspmm(80)
_spmm64 = _make_gspmm(64)


def _leaky(v):
    return jnp.where(v >= 0, v, 0.01 * v)


def _t1_body(ya, yb, dia, dib, doa, dob, W1, b1, Wg, bg,
             feat_o, rsqout16_o, invdeg_o, rsqin_o):
    deg_in = jnp.maximum(dia[:, 0:1] + dib[:, 0:1], 1.0)
    deg_out = jnp.maximum(doa[:, 0:1] + dob[:, 0:1], 1.0)
    invdeg = 1.0 / deg_in
    rsq_in = jnp.sqrt(invdeg)
    rsq_out = lax.rsqrt(deg_out)
    m1 = (ya[...] + yb[...]) * invdeg
    h1 = _leaky(jnp.dot(m1, W1[...],
                        preferred_element_type=_F32) + b1[...])
    feat_o[...] = jnp.dot(h1, Wg[...],
                          preferred_element_type=_F32) * rsq_out
    rsqout16_o[...] = jnp.broadcast_to(rsq_out, (N, 16))
    invdeg_o[...] = invdeg
    rsqin_o[...] = rsq_in


_t1 = pl.pallas_call(
    _t1_body,
    out_shape=(jax.ShapeDtypeStruct((N, 128), _F32),
               jax.ShapeDtypeStruct((N, 16), _F32),
               jax.ShapeDtypeStruct((N, 1), _F32),
               jax.ShapeDtypeStruct((N, 1), _F32)),
)


def _t2_body(ya, yb, rsqin, rsqout16, bg, W2, g3_o):
    agg2 = (ya[...] + yb[...]) * rsqin[...] + bg[...]
    h2 = _leaky(agg2)
    g3 = jnp.dot(h2, W2[...], preferred_element_type=_F32)
    g3_o[...] = jnp.concatenate([g3, rsqout16[...]], axis=1)


_t2 = pl.pallas_call(
    _t2_body,
    out_shape=jax.ShapeDtypeStruct((N, 80), _F32),
)


def _t3_body(ya, yb, invdeg, rsqin, b2, Wg2, bg2, W3, g5_o):
    h3 = (ya[:, :64] + yb[:, :64]) * invdeg[...] + b2[...]
    pooled = jnp.mean(h3, axis=0, keepdims=True)
    q = jnp.dot(pooled, Wg2[...], preferred_element_type=_F32)
    alpha = (ya[:, 64:65] + yb[:, 64:65]) * rsqin[...]
    h4 = _leaky(alpha * q + bg2[...])
    g5_o[...] = jnp.dot(h4, W3[...], preferred_element_type=_F32)


_t3 = pl.pallas_call(
    _t3_body,
    out_shape=jax.ShapeDtypeStruct((N, 64), _F32),
)


def _t4_body(ya, yb, invdeg, b3, out_o):
    out_o[...] = (ya[...] + yb[...]) * invdeg[...] + b3[...]


_t4 = pl.pallas_call(
    _t4_body,
    out_shape=jax.ShapeDtypeStruct((N, 64), _F32),
)


def kernel(x, edge_index, W1, b1, Wg, bg, W2, b2, Wg2, bg2, W3, b3):
    src_r = edge_index[0].reshape(NW, NCHUNK, CHUNK)
    dst_r = edge_index[1].reshape(NW, NCHUNK, CHUNK)

    din, dout = _deg_kernel(src_r, dst_r)
    y1 = _spmm128(x, src_r, dst_r)
    feat2, rsqout16, invdeg, rsqin = _t1(
        y1[0], y1[1], din[0], din[1], dout[0], dout[1], W1, b1, Wg, bg)
    y2 = _spmm128(feat2, src_r, dst_r)
    g3ext = _t2(y2[0], y2[1], rsqin, rsqout16, bg, W2)
    y3 = _spmm80(g3ext, src_r, dst_r)
    g5 = _t3(y3[0], y3[1], invdeg, rsqin, b2, Wg2, bg2, W3)
    y5 = _spmm64(g5, src_r, dst_r)
    out = _t4(y5[0], y5[1], invdeg, b3)
    return out


# speedup vs baseline: 17.2404x; 1.1719x over previous
import functools

import jax
import jax.numpy as jnp
from jax import lax
from jax.experimental import pallas as pl
from jax.experimental.pallas import tpu as pltpu
from jax.experimental.pallas import tpu_sc as plsc

N = 10000
E = 320000
NC = 2
NS = 16
NW = NC * NS
EPW = E // NW
CHUNK = 40
NCHUNK = EPW // CHUNK
RPT = N // NS
NBUF = 5
ZBR = 25

_F32 = jnp.float32


def _fill2d(ref, rows, cols, value):
    v = jnp.full((16,), value, _F32)
    nb = cols // 16

    def body(r, _):
        for k in range(nb):
            ref[r, pl.ds(16 * k, 16)] = v
        return 0

    lax.fori_loop(0, rows, body, 0)


def _zero_acc(zb, acc, s):
    _fill2d(zb, ZBR, zb.shape[1], 0.0)

    def body(k, _):
        pltpu.sync_copy(zb, acc.at[pl.ds(s * RPT + k * ZBR, ZBR)])
        return 0

    lax.fori_loop(0, RPT // ZBR, body, 0)


def _make_gspmm(dw):
    mesh = plsc.VectorSubcoreMesh(core_axis_name="c", subcore_axis_name="s")
    scratch = [
        pltpu.VMEM((NCHUNK, CHUNK), jnp.int32),
        pltpu.VMEM((NCHUNK, CHUNK), jnp.int32),
        pltpu.VMEM((ZBR, dw), _F32),
        pltpu.VMEM_SHARED((N, dw), _F32),
    ]
    scratch += [pltpu.VMEM((CHUNK, dw), _F32) for _ in range(NBUF)]
    scratch += [pltpu.SemaphoreType.DMA for _ in range(NBUF)]

    @functools.partial(
        pl.kernel, out_type=jax.ShapeDtypeStruct((NC, N, 128), _F32),
        mesh=mesh, scratch_types=tuple(scratch),
        compiler_params=pltpu.CompilerParams(use_tc_tiling_on_sc=False))
    def body(g_hbm, srcr_hbm, dstr_hbm, y_out, src_v, dst_v, zb, acc,
             *bufsem):
        bufs = bufsem[:NBUF]
        sems = bufsem[NBUF:]
        c = lax.axis_index("c")
        s = lax.axis_index("s")
        w = s * NC + c

        pltpu.sync_copy(srcr_hbm.at[w], src_v)
        pltpu.sync_copy(dstr_hbm.at[w], dst_v)
        _zero_acc(zb, acc, s)
        plsc.subcore_barrier()

        def fire(j, b):
            pltpu.async_copy(g_hbm.at[src_v.at[j]], bufs[b], sems[b])

        def drain_and_scatter(j, b):
            pltpu.make_async_copy(
                g_hbm.at[src_v.at[j]], bufs[b], sems[b]).wait()
            pltpu.sync_copy(bufs[b], acc.at[dst_v.at[j]], add=True)

        for b in range(NBUF):
            fire(b, b)

        def group(gi, _):
            j0 = gi * NBUF
            for b in range(NBUF):
                drain_and_scatter(j0 + b, b)
                fire(j0 + b + NBUF, b)
            return 0

        lax.fori_loop(0, NCHUNK // NBUF - 1, group, 0)
        for b in range(NBUF):
            drain_and_scatter(NCHUNK - NBUF + b, b)

        plsc.subcore_barrier()
        if dw == 128:
            pltpu.sync_copy(acc.at[pl.ds(s * RPT, RPT)],
                            y_out.at[c, pl.ds(s * RPT, RPT)])
        else:
            pltpu.sync_copy(acc.at[pl.ds(s * RPT, RPT)],
                            y_out.at[c, pl.ds(s * RPT, RPT), pl.ds(0, dw)])

    return body


def _make_deg():
    mesh = plsc.VectorSubcoreMesh(core_axis_name="c", subcore_axis_name="s")
    scratch = [
        pltpu.VMEM((NCHUNK, CHUNK), jnp.int32),
        pltpu.VMEM((NCHUNK, CHUNK), jnp.int32),
        pltpu.VMEM((CHUNK, 16), _F32),
        pltpu.VMEM((ZBR, 16), _F32),
        pltpu.VMEM_SHARED((N, 16), _F32),
        pltpu.VMEM_SHARED((N, 16), _F32),
        pltpu.SemaphoreType.DMA,
        pltpu.SemaphoreType.DMA,
    ]

    @functools.partial(
        pl.kernel, out_type=jax.ShapeDtypeStruct((NC, N, 128), _F32),
        mesh=mesh, scratch_types=tuple(scratch),
        compiler_params=pltpu.CompilerParams(use_tc_tiling_on_sc=False))
    def body(srcr_hbm, dstr_hbm, deg_out_hbm, src_v, dst_v,
             ones_v, zb, din, dout, sem1, sem2):
        c = lax.axis_index("c")
        s = lax.axis_index("s")
        w = s * NC + c

        pltpu.sync_copy(srcr_hbm.at[w], src_v)
        pltpu.sync_copy(dstr_hbm.at[w], dst_v)
        _fill2d(ones_v, CHUNK, 16, 1.0)
        _zero_acc(zb, din, s)
        _zero_acc(zb, dout, s)
        plsc.subcore_barrier()

        def fire(j):
            pltpu.async_copy(ones_v, din.at[dst_v.at[j]], sem1, add=True)
            pltpu.async_copy(ones_v, dout.at[src_v.at[j]], sem2, add=True)

        def drain(j):
            pltpu.make_async_copy(ones_v, din.at[dst_v.at[j]],
                                  sem1).wait()
            pltpu.make_async_copy(ones_v, dout.at[src_v.at[j]],
                                  sem2).wait()

        fire(0)

        def chunk(j, _):
            fire(j)
            drain(j - 1)
            return 0

        lax.fori_loop(1, NCHUNK, chunk, 0)
        drain(NCHUNK - 1)
        plsc.subcore_barrier()
        pltpu.sync_copy(din.at[pl.ds(s * RPT, RPT)],
                        deg_out_hbm.at[c, pl.ds(s * RPT, RPT),
                                       pl.ds(0, 16)])
        pltpu.sync_copy(dout.at[pl.ds(s * RPT, RPT)],
                        deg_out_hbm.at[c, pl.ds(s * RPT, RPT),
                                       pl.ds(16, 16)])

    return body


_deg_kernel = _make_deg()
_spmm128 = _make_gspmm(128)
_spmm80 = _make_gspmm(80)
_spmm64 = _make_gspmm(64)


def _leaky(v):
    return jnp.where(v >= 0, v, 0.01 * v)


def _t1_body(yp, degs, W1, b1, Wg, bg, feat_o, rsqout16_o, invdeg_o,
             rsqin_o):
    deg_in = jnp.maximum(degs[0, :, 0:1] + degs[1, :, 0:1], 1.0)
    deg_out = jnp.maximum(degs[0, :, 16:17] + degs[1, :, 16:17], 1.0)
    invdeg = 1.0 / deg_in
    rsq_in = jnp.sqrt(invdeg)
    rsq_out = lax.rsqrt(deg_out)
    m1 = (yp[0] + yp[1]) * invdeg
    h1 = _leaky(jnp.dot(m1, W1[...],
                        preferred_element_type=_F32) + b1[...])
    feat_o[...] = jnp.dot(h1, Wg[...],
                          preferred_element_type=_F32) * rsq_out
    rsqout16_o[...] = jnp.broadcast_to(rsq_out, (N, 16))
    invdeg_o[...] = invdeg
    rsqin_o[...] = rsq_in


_t1 = pl.pallas_call(
    _t1_body,
    out_shape=(jax.ShapeDtypeStruct((N, 128), _F32),
               jax.ShapeDtypeStruct((N, 16), _F32),
               jax.ShapeDtypeStruct((N, 1), _F32),
               jax.ShapeDtypeStruct((N, 1), _F32)),
)


def _t2_body(yp, rsqin, rsqout16, bg, W2, g3_o):
    agg2 = (yp[0] + yp[1]) * rsqin[...] + bg[...]
    h2 = _leaky(agg2)
    g3 = jnp.dot(h2, W2[...], preferred_element_type=_F32)
    g3_o[...] = jnp.concatenate([g3, rsqout16[...]], axis=1)


_t2 = pl.pallas_call(
    _t2_body,
    out_shape=jax.ShapeDtypeStruct((N, 80), _F32),
)


def _t3_body(yp, invdeg, rsqin, b2, Wg2, bg2, W3, g5_o):
    h3 = (yp[0, :, :64] + yp[1, :, :64]) * invdeg[...] + b2[...]
    pooled = jnp.mean(h3, axis=0, keepdims=True)
    q = jnp.dot(pooled, Wg2[...], preferred_element_type=_F32)
    alpha = (yp[0, :, 64:65] + yp[1, :, 64:65]) * rsqin[...]
    h4 = _leaky(alpha * q + bg2[...])
    g5_o[...] = jnp.dot(h4, W3[...], preferred_element_type=_F32)


_t3 = pl.pallas_call(
    _t3_body,
    out_shape=jax.ShapeDtypeStruct((N, 64), _F32),
)


def _t4_body(yp, invdeg, b3, out_o):
    out_o[...] = (yp[0, :, :64] + yp[1, :, :64]) * invdeg[...] + b3[...]


_t4 = pl.pallas_call(
    _t4_body,
    out_shape=jax.ShapeDtypeStruct((N, 64), _F32),
)


def kernel(x, edge_index, W1, b1, Wg, bg, W2, b2, Wg2, bg2, W3, b3):
    src_r = edge_index[0].reshape(NW, NCHUNK, CHUNK)
    dst_r = edge_index[1].reshape(NW, NCHUNK, CHUNK)

    degs = _deg_kernel(src_r, dst_r)
    y1 = _spmm128(x, src_r, dst_r)
    feat2, rsqout16, invdeg, rsqin = _t1(y1, degs, W1, b1, Wg, bg)
    y2 = _spmm128(feat2, src_r, dst_r)
    g3ext = _t2(y2, rsqin, rsqout16, bg, W2)
    y3 = _spmm80(g3ext, src_r, dst_r)
    g5 = _t3(y3, invdeg, rsqin, b2, Wg2, bg2, W3)
    y5 = _spmm64(g5, src_r, dst_r)
    out = _t4(y5, invdeg, b3)
    return out
